# baseline (speedup 1.0000x reference)
import jax
import jax.numpy as jnp
from jax import lax
from jax.experimental import pallas as pl
from jax.experimental.pallas import tpu as pltpu
from jax.experimental.pallas import tpu_sc as plsc

B, S, D = 4096, 50, 64
NC, NS, L = 2, 16, 16
NW = NC * NS
R = B // NW
G = R // L
C = D // L


def _body(in_hbm, par_hbm, out_hbm, in_v, p_v, p0_v, out_v, p0_sh,
          sem_in, sem_p):
    wid = lax.axis_index("s") * NC + lax.axis_index("c")
    base = wid * R
    cp_in = pltpu.async_copy(in_hbm.at[:, pl.ds(base, R)], in_v, sem_in)

    iota = lax.iota(jnp.int32, L)
    zero = jnp.zeros((L,), jnp.int32)

    @pl.when(lax.axis_index("s") == 0)
    def _():
        cp_p = pltpu.async_copy(par_hbm.at[:, pl.ds(0, 128)], p_v, sem_p)
        cp_p.wait()
        for c in range(C):
            p0_v[pl.ds(c * L, L)] = plsc.load_gather(p_v, [iota + c * L, zero])
        pltpu.sync_copy(p0_v, p0_sh)

    cp_in.wait()

    def col(l, accs):
        return tuple(a + in_v[l, pl.ds(g * L, L)] for g, a in enumerate(accs))

    accs = lax.fori_loop(0, S, col,
                         tuple(jnp.zeros((L,), jnp.float32) for _ in range(G)),
                         unroll=5)

    plsc.subcore_barrier()
    pltpu.sync_copy(p0_sh, p0_v)

    def emit(d, _):
        pd = plsc.load_gather(p0_v, [jnp.full((L,), 0, jnp.int32) + d])
        for g in range(G):
            out_v[d, pl.ds(g * L, L)] = pd * accs[g]
        return _

    lax.fori_loop(0, D, emit, 0, unroll=4)
    pltpu.sync_copy(out_v, out_hbm.at[:, pl.ds(base, R)])


@jax.jit
def kernel(inputs, params):
    mesh = plsc.VectorSubcoreMesh(
        core_axis_name="c", subcore_axis_name="s",
        num_cores=NC, num_subcores=NS,
    )
    out_t = pl.kernel(
        _body,
        out_type=jax.ShapeDtypeStruct((D, B), jnp.float32),
        mesh=mesh,
        compiler_params=pltpu.CompilerParams(
            needs_layout_passes=False, use_tc_tiling_on_sc=True),
        scratch_types=[
            pltpu.VMEM((S, R), jnp.float32),
            pltpu.VMEM((D, 128), jnp.float32),
            pltpu.VMEM((D,), jnp.float32),
            pltpu.VMEM((D, R), jnp.float32),
            pltpu.VMEM_SHARED((D,), jnp.float32),
            pltpu.SemaphoreType.DMA,
            pltpu.SemaphoreType.DMA,
        ],
    )(inputs.T, params.T)
    return out_t.T[:, None, :]


# speedup vs baseline: 1.0030x; 1.0030x over previous
import jax
import jax.numpy as jnp
from jax import lax
from jax.experimental import pallas as pl
from jax.experimental.pallas import tpu as pltpu
from jax.experimental.pallas import tpu_sc as plsc

B, S, D = 4096, 50, 64
NC, NS, L = 1, 16, 16
NW = NC * NS
R = B // NW
G = R // L
C = D // L


def _body(in_hbm, par_hbm, out_hbm, in_v, p_v, p0_v, out_v, p0_sh,
          sem_in, sem_p):
    wid = lax.axis_index("s") * NC + lax.axis_index("c")
    base = wid * R
    cp_in = pltpu.async_copy(in_hbm.at[:, pl.ds(base, R)], in_v, sem_in)

    iota = lax.iota(jnp.int32, L)
    zero = jnp.zeros((L,), jnp.int32)

    @pl.when(wid == 0)
    def _():
        cp_p = pltpu.async_copy(par_hbm.at[:, pl.ds(0, 128)], p_v, sem_p)
        cp_p.wait()
        for c in range(C):
            p0_v[pl.ds(c * L, L)] = plsc.load_gather(p_v, [iota + c * L, zero])
        pltpu.sync_copy(p0_v, p0_sh)

    cp_in.wait()

    def col(l, accs):
        return tuple(a + in_v[l, pl.ds(g * L, L)] for g, a in enumerate(accs))

    accs = lax.fori_loop(0, S, col,
                         tuple(jnp.zeros((L,), jnp.float32) for _ in range(G)),
                         unroll=5)

    plsc.subcore_barrier()
    pltpu.sync_copy(p0_sh, p0_v)

    def emit(d, _):
        pd = plsc.load_gather(p0_v, [jnp.full((L,), 0, jnp.int32) + d])
        for g in range(G):
            out_v[d, pl.ds(g * L, L)] = pd * accs[g]
        return _

    lax.fori_loop(0, D, emit, 0, unroll=4)
    pltpu.sync_copy(out_v, out_hbm.at[:, pl.ds(base, R)])


@jax.jit
def kernel(inputs, params):
    mesh = plsc.VectorSubcoreMesh(
        core_axis_name="c", subcore_axis_name="s",
        num_cores=NC, num_subcores=NS,
    )
    out_t = pl.kernel(
        _body,
        out_type=jax.ShapeDtypeStruct((D, B), jnp.float32),
        mesh=mesh,
        compiler_params=pltpu.CompilerParams(
            needs_layout_passes=False, use_tc_tiling_on_sc=True),
        scratch_types=[
            pltpu.VMEM((S, R), jnp.float32),
            pltpu.VMEM((D, 128), jnp.float32),
            pltpu.VMEM((D,), jnp.float32),
            pltpu.VMEM((D, R), jnp.float32),
            pltpu.VMEM_SHARED((D,), jnp.float32),
            pltpu.SemaphoreType.DMA,
            pltpu.SemaphoreType.DMA,
        ],
    )(inputs.T, params.T)
    return out_t.T[:, None, :]
